# baseline (speedup 1.0000x reference)
import functools

import jax
import jax.numpy as jnp
from jax.experimental import pallas as pl
import jax.experimental.pallas.tpu as pltpu

CHUNK = 8192
NBUF = 4
TAIL = (4096, 2048, 1024, 512, 512)


def _softmax_colsum(w16, b_col, x_f32):
    lt = jax.lax.dot_general(
        w16, x_f32.astype(jnp.bfloat16),
        (((1,), (1,)), ((), ())),
        preferred_element_type=jnp.float32,
    ) + b_col
    m = jnp.max(lt, axis=0, keepdims=True)
    e = jnp.exp(lt - m)
    s = jnp.sum(e, axis=0, keepdims=True)
    p = e / s
    return jnp.sum(p, axis=1, keepdims=True)


def _router_kernel(x_hbm, w_ref, b_ref, ssm_ref, msm_ref, out_ref,
                   buf_ref, tail_ref, sem, tsem, *, grid, inv_n):
    tail_off = grid * CHUNK

    def copy(idx, slot):
        return pltpu.make_async_copy(
            x_hbm.at[pl.ds(idx * CHUNK, CHUNK), :],
            buf_ref.at[slot],
            sem.at[slot],
        )

    def tail_copy(k):
        off = sum(TAIL[:k])
        return pltpu.make_async_copy(
            x_hbm.at[pl.ds(tail_off + off, TAIL[k]), :],
            tail_ref.at[pl.ds(off, TAIL[k]), :],
            tsem.at[k],
        )

    for k in range(NBUF - 1):
        copy(k, k).start()

    w16 = w_ref[...].astype(jnp.bfloat16)
    b_col = b_ref[...].T

    def step(i, acc):
        slot = jax.lax.rem(i, NBUF)
        nxt = i + NBUF - 1

        @pl.when(nxt < grid)
        def _prefetch():
            copy(nxt, jax.lax.rem(nxt, NBUF)).start()

        @pl.when(i == grid - NBUF)
        def _queue_tail():
            for k in range(len(TAIL)):
                tail_copy(k).start()

        copy(i, slot).wait()
        return acc + _softmax_colsum(w16, b_col, buf_ref[slot])

    acc = jax.lax.fori_loop(
        0, grid, step, jnp.zeros((w_ref.shape[0], 1), jnp.float32))

    for k in range(len(TAIL)):
        off = sum(TAIL[:k])
        tail_copy(k).wait()
        acc = acc + _softmax_colsum(
            w16, b_col, tail_ref[pl.ds(off, TAIL[k]), :])

    avg = acc * inv_n
    nssm = ssm_ref.shape[0]
    out_ref[0:nssm, :] = ssm_ref[...] * avg[0:nssm, :]
    out_ref[nssm:, :] = msm_ref[...] * avg[nssm:, :]


def kernel(input_stream, ssm_slots, msm_slots, router_w, router_b,
           compress_w, compress_b):
    del compress_w, compress_b
    n, d = input_stream.shape
    nslots = router_w.shape[0]
    grid = (n - sum(TAIL)) // CHUNK

    out = pl.pallas_call(
        functools.partial(_router_kernel, grid=grid, inv_n=1.0 / n),
        in_specs=[
            pl.BlockSpec(memory_space=pl.ANY),
            pl.BlockSpec((nslots, d), lambda: (0, 0)),
            pl.BlockSpec((1, nslots), lambda: (0, 0)),
            pl.BlockSpec(ssm_slots.shape, lambda: (0, 0)),
            pl.BlockSpec(msm_slots.shape, lambda: (0, 0)),
        ],
        out_specs=pl.BlockSpec((nslots, d), lambda: (0, 0)),
        out_shape=jax.ShapeDtypeStruct((nslots, d), jnp.float32),
        scratch_shapes=[
            pltpu.VMEM((NBUF, CHUNK, d), jnp.float32),
            pltpu.VMEM((sum(TAIL), d), jnp.float32),
            pltpu.SemaphoreType.DMA((NBUF,)),
            pltpu.SemaphoreType.DMA((len(TAIL),)),
        ],
    )(input_stream, router_w, router_b.reshape(1, nslots),
      ssm_slots, msm_slots)
    return out


# speedup vs baseline: 1.0831x; 1.0831x over previous
import functools

import jax
import jax.numpy as jnp
from jax.experimental import pallas as pl
import jax.experimental.pallas.tpu as pltpu

CHUNK = 8192


def _router_kernel(x_ref, w_ref, b_ref, ssm_ref, msm_ref, out_ref, acc_ref,
                   *, grid, inv_n):
    i = pl.program_id(0)
    lt = jax.lax.dot_general(
        w_ref[...].astype(jnp.bfloat16), x_ref[...].astype(jnp.bfloat16),
        (((1,), (1,)), ((), ())),
        preferred_element_type=jnp.float32,
    ) + b_ref[...].T
    m = jnp.max(lt, axis=0, keepdims=True)
    e = jnp.exp(lt - m)
    s = jnp.sum(e, axis=0, keepdims=True)
    p = e / s
    part = jnp.sum(p, axis=1, keepdims=True)

    @pl.when(i == 0)
    def _init():
        acc_ref[...] = part

    @pl.when(i > 0)
    def _acc():
        acc_ref[...] += part

    @pl.when(i == grid - 1)
    def _finish():
        nssm = ssm_ref.shape[0]
        avg = acc_ref[...] * inv_n
        out_ref[0:nssm, :] = ssm_ref[...] * avg[0:nssm, :]
        out_ref[nssm:, :] = msm_ref[...] * avg[nssm:, :]


def kernel(input_stream, ssm_slots, msm_slots, router_w, router_b,
           compress_w, compress_b):
    del compress_w, compress_b
    n, d = input_stream.shape
    nslots = router_w.shape[0]
    grid = n // CHUNK

    out = pl.pallas_call(
        functools.partial(_router_kernel, grid=grid, inv_n=1.0 / n),
        grid=(grid,),
        in_specs=[
            pl.BlockSpec((CHUNK, d), lambda i: (i, 0)),
            pl.BlockSpec((nslots, d), lambda i: (0, 0)),
            pl.BlockSpec((1, nslots), lambda i: (0, 0)),
            pl.BlockSpec(ssm_slots.shape, lambda i: (0, 0)),
            pl.BlockSpec(msm_slots.shape, lambda i: (0, 0)),
        ],
        out_specs=pl.BlockSpec((nslots, d), lambda i: (0, 0)),
        out_shape=jax.ShapeDtypeStruct((nslots, d), jnp.float32),
        scratch_shapes=[pltpu.VMEM((nslots, 1), jnp.float32)],
    )(input_stream, router_w, router_b.reshape(1, nslots),
      ssm_slots, msm_slots)
    return out


# speedup vs baseline: 1.0897x; 1.0061x over previous
import functools

import jax
import jax.numpy as jnp
from jax.experimental import pallas as pl
import jax.experimental.pallas.tpu as pltpu

CHUNK = 8192
TAIL = (4096, 2048, 1024, 512, 512)


def _softmax_colsum(w16, b_col, x_f32):
    lt = jax.lax.dot_general(
        w16, x_f32.astype(jnp.bfloat16),
        (((1,), (1,)), ((), ())),
        preferred_element_type=jnp.float32,
    ) + b_col
    m = jnp.max(lt, axis=0, keepdims=True)
    e = jnp.exp(lt - m)
    s = jnp.sum(e, axis=0, keepdims=True)
    p = e / s
    return jnp.sum(p, axis=1, keepdims=True)


def _router_kernel(x_ref, x_hbm, w_ref, b_ref, ssm_ref, msm_ref, out_ref,
                   acc_ref, tail_ref, tsem, *, grid, inv_n):
    i = pl.program_id(0)
    tail_off = grid * CHUNK

    def tail_copy(k):
        off = sum(TAIL[:k])
        return pltpu.make_async_copy(
            x_hbm.at[pl.ds(tail_off + off, TAIL[k]), :],
            tail_ref.at[pl.ds(off, TAIL[k]), :],
            tsem.at[k],
        )

    w16 = w_ref[...].astype(jnp.bfloat16)
    b_col = b_ref[...].T
    part = _softmax_colsum(w16, b_col, x_ref[...])

    @pl.when(i == 0)
    def _init():
        acc_ref[...] = part

    @pl.when(i > 0)
    def _acc():
        acc_ref[...] += part

    @pl.when(i == grid - 2)
    def _queue_tail():
        for k in range(len(TAIL)):
            tail_copy(k).start()

    @pl.when(i == grid - 1)
    def _finish():
        acc = acc_ref[...]
        for k in range(len(TAIL)):
            off = sum(TAIL[:k])
            tail_copy(k).wait()
            acc = acc + _softmax_colsum(
                w16, b_col, tail_ref[pl.ds(off, TAIL[k]), :])
        avg = acc * inv_n
        nssm = ssm_ref.shape[0]
        out_ref[0:nssm, :] = ssm_ref[...] * avg[0:nssm, :]
        out_ref[nssm:, :] = msm_ref[...] * avg[nssm:, :]


def kernel(input_stream, ssm_slots, msm_slots, router_w, router_b,
           compress_w, compress_b):
    del compress_w, compress_b
    n, d = input_stream.shape
    nslots = router_w.shape[0]
    grid = (n - sum(TAIL)) // CHUNK

    out = pl.pallas_call(
        functools.partial(_router_kernel, grid=grid, inv_n=1.0 / n),
        grid=(grid,),
        in_specs=[
            pl.BlockSpec((CHUNK, d), lambda i: (i, 0)),
            pl.BlockSpec(memory_space=pl.ANY),
            pl.BlockSpec((nslots, d), lambda i: (0, 0)),
            pl.BlockSpec((1, nslots), lambda i: (0, 0)),
            pl.BlockSpec(ssm_slots.shape, lambda i: (0, 0)),
            pl.BlockSpec(msm_slots.shape, lambda i: (0, 0)),
        ],
        out_specs=pl.BlockSpec((nslots, d), lambda i: (0, 0)),
        out_shape=jax.ShapeDtypeStruct((nslots, d), jnp.float32),
        scratch_shapes=[
            pltpu.VMEM((nslots, 1), jnp.float32),
            pltpu.VMEM((sum(TAIL), d), jnp.float32),
            pltpu.SemaphoreType.DMA((len(TAIL),)),
        ],
    )(input_stream, input_stream, router_w, router_b.reshape(1, nslots),
      ssm_slots, msm_slots)
    return out
